# baseline (speedup 1.0000x reference)
import jax
import jax.numpy as jnp
from jax import lax
from jax.experimental import pallas as pl
from jax.experimental.pallas import tpu as pltpu

_C = 512
_NBUF = 8


def _body(gid_ref, table_ref, tok_hbm, out_hbm, *scratch):
    bufs = scratch[:_NBUF]
    in_sems = scratch[_NBUF:2 * _NBUF]
    out_sems = scratch[2 * _NBUF:3 * _NBUF]
    rows = tok_hbm.shape[0]
    nchunk = rows // _C
    gid = gid_ref[0]
    vec = table_ref[gid, :]

    for b in range(_NBUF):
        pltpu.make_async_copy(
            tok_hbm.at[pl.ds(b * _C, _C)], bufs[b], in_sems[b]).start()

    def _step(g, b):
        pltpu.make_async_copy(
            tok_hbm.at[pl.ds(0, _C)], bufs[b], in_sems[b]).wait()

        bufs[b][...] = bufs[b][...] + vec[None, :]

        pltpu.make_async_copy(
            bufs[b], out_hbm.at[pl.ds(g * _C, _C)], out_sems[b]).start()

        @pl.when(g + _NBUF < nchunk)
        def _():
            pltpu.make_async_copy(
                bufs[b], out_hbm.at[pl.ds(0, _C)], out_sems[b]).wait()
            pltpu.make_async_copy(
                tok_hbm.at[pl.ds((g + _NBUF) * _C, _C)],
                bufs[b], in_sems[b]).start()

    def _outer(i, carry):
        for b in range(_NBUF):
            _step(i * _NBUF + b, b)
        return carry

    lax.fori_loop(0, nchunk // _NBUF, _outer, 0)

    for b in range(_NBUF):
        pltpu.make_async_copy(
            bufs[b], out_hbm.at[pl.ds(0, _C)], out_sems[b]).wait()


def kernel(tokens, group_id, group_id_vecs):
    b, s, d = tokens.shape
    rows = b * s
    tok2d = tokens.reshape(rows, d)
    gid = jnp.asarray(group_id, jnp.int32).reshape((1,))
    out = pl.pallas_call(
        _body,
        grid_spec=pltpu.PrefetchScalarGridSpec(
            num_scalar_prefetch=1,
            grid=(1,),
            in_specs=[
                pl.BlockSpec(memory_space=pltpu.VMEM),
                pl.BlockSpec(memory_space=pltpu.HBM),
            ],
            out_specs=pl.BlockSpec(memory_space=pltpu.HBM),
            scratch_shapes=(
                [pltpu.VMEM((_C, d), jnp.float32)] * _NBUF
                + [pltpu.SemaphoreType.DMA] * (2 * _NBUF)
            ),
        ),
        out_shape=jax.ShapeDtypeStruct((rows, d), tokens.dtype),
    )(gid, group_id_vecs, tok2d)
    return out.reshape(b, s, d)
